# initial kernel scaffold (unmeasured)
import jax
import jax.numpy as jnp
from jax import lax
from jax.experimental import pallas as pl
from jax.experimental.pallas import tpu as pltpu

N_DEV = 16


def kernel(A, B):
    m_per, k = A.shape
    _, n = B.shape

    def body(a_ref, b_ref, out_ref, comm_ref, out_vmem,
             send_sem, recv_sems, copy_sem, credit_sem):
        my = lax.axis_index("i")
        left = lax.rem(my + N_DEV - 1, N_DEV)
        right = lax.rem(my + 1, N_DEV)

        barrier_sem = pltpu.get_barrier_semaphore()
        for nbr in (left, right):
            pl.semaphore_signal(
                barrier_sem, inc=1,
                device_id=(nbr,), device_id_type=pl.DeviceIdType.MESH,
            )
        pl.semaphore_wait(barrier_sem, 2)

        stage = pltpu.make_async_copy(a_ref, comm_ref.at[0], copy_sem)
        stage.start()
        stage.wait()

        out_vmem[...] = jnp.dot(a_ref[...], b_ref[...],
                                preferred_element_type=jnp.float32)
        cp = pltpu.make_async_copy(
            out_vmem, out_ref.at[pl.ds(my * m_per, m_per)], copy_sem)
        cp.start()
        cp.wait()

        for h in range(N_DEV - 1):
            send_slot = h % 2
            recv_slot = (h + 1) % 2
            if h >= 1:
                pl.semaphore_wait(credit_sem, 1)
            rdma = pltpu.make_async_remote_copy(
                src_ref=comm_ref.at[send_slot],
                dst_ref=comm_ref.at[recv_slot],
                send_sem=send_sem,
                recv_sem=recv_sems.at[h],
                device_id=(right,),
                device_id_type=pl.DeviceIdType.MESH,
            )
            rdma.start()
            rdma.wait_send()
            if h <= N_DEV - 3:
                pl.semaphore_signal(
                    credit_sem, inc=1,
                    device_id=(left,), device_id_type=pl.DeviceIdType.MESH,
                )
            rdma.wait_recv()
            origin = lax.rem(my + N_DEV - h - 1, N_DEV)
            out_vmem[...] = jnp.dot(comm_ref[recv_slot], b_ref[...],
                                    preferred_element_type=jnp.float32)
            cp = pltpu.make_async_copy(
                out_vmem, out_ref.at[pl.ds(origin * m_per, m_per)], copy_sem)
            cp.start()
            cp.wait()

    out_shape = jax.ShapeDtypeStruct((N_DEV * m_per, n), jnp.float32)
    return pl.pallas_call(
        body,
        out_shape=out_shape,
        in_specs=[
            pl.BlockSpec(memory_space=pltpu.VMEM),
            pl.BlockSpec(memory_space=pltpu.VMEM),
        ],
        out_specs=pl.BlockSpec(memory_space=pltpu.ANY),
        scratch_shapes=[
            pltpu.VMEM((2, m_per, k), jnp.float32),
            pltpu.VMEM((m_per, n), jnp.float32),
            pltpu.SemaphoreType.DMA,
            pltpu.SemaphoreType.DMA((N_DEV - 1,)),
            pltpu.SemaphoreType.DMA,
            pltpu.SemaphoreType.REGULAR,
        ],
        compiler_params=pltpu.CompilerParams(
            collective_id=0,
            vmem_limit_bytes=100 * 1024 * 1024,
        ),
    )(A, B)


# baseline (device time: 1803249 ns/iter reference)
import jax
import jax.numpy as jnp
from jax import lax
from jax.experimental import pallas as pl
from jax.experimental.pallas import tpu as pltpu

N_DEV = 16


def kernel(A, B):
    m_per, k = A.shape
    _, n = B.shape

    def body(a_ref, b_ref, out_ref, comm_ref, out_vmem,
             send_sem, recv_sems, copy_sem, credit_sem):
        my = lax.axis_index("i")
        left = lax.rem(my + N_DEV - 1, N_DEV)
        right = lax.rem(my + 1, N_DEV)

        barrier_sem = pltpu.get_barrier_semaphore()
        for nbr in (left, right):
            pl.semaphore_signal(
                barrier_sem, inc=1,
                device_id=(nbr,), device_id_type=pl.DeviceIdType.MESH,
            )
        pl.semaphore_wait(barrier_sem, 2)

        stage = pltpu.make_async_copy(a_ref, comm_ref.at[0], copy_sem)
        stage.start()
        stage.wait()

        out_vmem[...] = jnp.dot(a_ref[...], b_ref[...],
                                preferred_element_type=jnp.float32)
        cp = pltpu.make_async_copy(
            out_vmem, out_ref.at[pl.ds(my * m_per, m_per)], copy_sem)
        cp.start()
        cp.wait()

        def hop(h, carry):
            send_slot = lax.rem(h, 2)
            recv_slot = lax.rem(h + 1, 2)

            @pl.when(h >= 1)
            def _():
                pl.semaphore_wait(credit_sem, 1)

            rdma = pltpu.make_async_remote_copy(
                src_ref=comm_ref.at[send_slot],
                dst_ref=comm_ref.at[recv_slot],
                send_sem=send_sem,
                recv_sem=recv_sems.at[h],
                device_id=(right,),
                device_id_type=pl.DeviceIdType.MESH,
            )
            rdma.start()
            rdma.wait_send()

            @pl.when(h <= N_DEV - 3)
            def _():
                pl.semaphore_signal(
                    credit_sem, inc=1,
                    device_id=(left,), device_id_type=pl.DeviceIdType.MESH,
                )

            rdma.wait_recv()
            origin = lax.rem(my + N_DEV - h - 1, N_DEV)
            out_vmem[...] = jnp.dot(comm_ref[recv_slot], b_ref[...],
                                    preferred_element_type=jnp.float32)
            cp = pltpu.make_async_copy(
                out_vmem, out_ref.at[pl.ds(origin * m_per, m_per)], copy_sem)
            cp.start()
            cp.wait()
            return carry

        lax.fori_loop(0, N_DEV - 1, hop, 0)

    out_shape = jax.ShapeDtypeStruct((N_DEV * m_per, n), jnp.float32)
    return pl.pallas_call(
        body,
        out_shape=out_shape,
        in_specs=[
            pl.BlockSpec(memory_space=pltpu.VMEM),
            pl.BlockSpec(memory_space=pltpu.VMEM),
        ],
        out_specs=pl.BlockSpec(memory_space=pl.ANY),
        scratch_shapes=[
            pltpu.VMEM((2, m_per, k), jnp.float32),
            pltpu.VMEM((m_per, n), jnp.float32),
            pltpu.SemaphoreType.DMA,
            pltpu.SemaphoreType.DMA((N_DEV - 1,)),
            pltpu.SemaphoreType.DMA,
            pltpu.SemaphoreType.REGULAR,
        ],
        compiler_params=pltpu.CompilerParams(
            collective_id=0,
            vmem_limit_bytes=100 * 1024 * 1024,
        ),
    )(A, B)


# device time: 586137 ns/iter; 3.0765x vs baseline; 3.0765x over previous
import jax
import jax.numpy as jnp
from jax import lax
from jax.experimental import pallas as pl
from jax.experimental.pallas import tpu as pltpu

N_DEV = 16
CW_ROUNDS = N_DEV // 2
CCW_ROUNDS = N_DEV // 2 - 1


def kernel(A, B):
    m_per, k = A.shape
    _, n = B.shape
    m_half = m_per // 2

    def body(a_ref, b_ref, out_ref, cw_buf, ccw_buf, b16_ref, slab,
             cw_send_sem, ccw_send_sem, cw_recv_sems, ccw_recv_sems,
             copy_sem, credit_cw, credit_ccw):
        my = lax.axis_index("i")
        left = lax.rem(my + N_DEV - 1, N_DEV)
        right = lax.rem(my + 1, N_DEV)

        barrier_sem = pltpu.get_barrier_semaphore()
        for nbr in (left, right):
            pl.semaphore_signal(
                barrier_sem, inc=1,
                device_id=(nbr,), device_id_type=pl.DeviceIdType.MESH,
            )
        pl.semaphore_wait(barrier_sem, 2)

        av = a_ref[...].astype(jnp.bfloat16)
        cw_buf[1, ...] = av
        ccw_buf[1, ...] = av
        bcp = pltpu.make_async_copy(b_ref, slab, copy_sem)
        bcp.start()
        bcp.wait()
        b16_ref[...] = slab[...].astype(jnp.bfloat16)

        def emit_chunk(chunk, origin):
            row0 = origin * m_per
            for half in range(2):
                slab[...] = jnp.dot(
                    chunk[pl.ds(half * m_half, m_half), :], b16_ref[...],
                    preferred_element_type=jnp.float32)
                cp = pltpu.make_async_copy(
                    slab,
                    out_ref.at[pl.ds(row0 + half * m_half, m_half)],
                    copy_sem)
                cp.start()
                cp.wait()

        def hop(r, carry):
            sslot = lax.rem(r + 1, 2)
            rslot = lax.rem(r, 2)

            @pl.when(r >= 1)
            def _():
                pl.semaphore_wait(credit_cw, 1)

            cw_rdma = pltpu.make_async_remote_copy(
                src_ref=cw_buf.at[sslot],
                dst_ref=cw_buf.at[rslot],
                send_sem=cw_send_sem,
                recv_sem=cw_recv_sems.at[r],
                device_id=(right,),
                device_id_type=pl.DeviceIdType.MESH,
            )
            cw_rdma.start()

            ccw_rdma = pltpu.make_async_remote_copy(
                src_ref=ccw_buf.at[sslot],
                dst_ref=ccw_buf.at[rslot],
                send_sem=ccw_send_sem,
                recv_sem=ccw_recv_sems.at[r],
                device_id=(left,),
                device_id_type=pl.DeviceIdType.MESH,
            )

            @pl.when(r <= CCW_ROUNDS - 1)
            def _():
                @pl.when(r >= 1)
                def _():
                    pl.semaphore_wait(credit_ccw, 1)
                ccw_rdma.start()

            emit_chunk(cw_buf.at[sslot], lax.rem(my + N_DEV - r, N_DEV))

            @pl.when(r >= 1)
            def _():
                emit_chunk(ccw_buf.at[sslot], lax.rem(my + r, N_DEV))

            cw_rdma.wait_send()

            @pl.when(r <= CW_ROUNDS - 2)
            def _():
                pl.semaphore_signal(
                    credit_cw, inc=1,
                    device_id=(left,), device_id_type=pl.DeviceIdType.MESH,
                )

            @pl.when(r <= CCW_ROUNDS - 1)
            def _():
                ccw_rdma.wait_send()

                @pl.when(r <= CCW_ROUNDS - 2)
                def _():
                    pl.semaphore_signal(
                        credit_ccw, inc=1,
                        device_id=(right,),
                        device_id_type=pl.DeviceIdType.MESH,
                    )

                ccw_rdma.wait_recv()

            cw_rdma.wait_recv()
            return carry

        lax.fori_loop(0, CW_ROUNDS, hop, 0)

        emit_chunk(cw_buf.at[1], lax.rem(my + CW_ROUNDS, N_DEV))

    out_shape = jax.ShapeDtypeStruct((N_DEV * m_per, n), jnp.float32)
    return pl.pallas_call(
        body,
        out_shape=out_shape,
        in_specs=[
            pl.BlockSpec(memory_space=pltpu.VMEM),
            pl.BlockSpec(memory_space=pl.ANY),
        ],
        out_specs=pl.BlockSpec(memory_space=pl.ANY),
        scratch_shapes=[
            pltpu.VMEM((2, m_per, k), jnp.bfloat16),
            pltpu.VMEM((2, m_per, k), jnp.bfloat16),
            pltpu.VMEM((k, n), jnp.bfloat16),
            pltpu.VMEM((m_per // 2, n), jnp.float32),
            pltpu.SemaphoreType.DMA,
            pltpu.SemaphoreType.DMA,
            pltpu.SemaphoreType.DMA((CW_ROUNDS,)),
            pltpu.SemaphoreType.DMA((CW_ROUNDS,)),
            pltpu.SemaphoreType.DMA,
            pltpu.SemaphoreType.REGULAR,
            pltpu.SemaphoreType.REGULAR,
        ],
        compiler_params=pltpu.CompilerParams(
            collective_id=0,
            vmem_limit_bytes=100 * 1024 * 1024,
        ),
    )(A, B)
